# Initial kernel scaffold; baseline (speedup 1.0000x reference)
import jax
import jax.numpy as jnp
from jax.experimental import pallas as pl


def kernel(cat_idx_sex, cat_idx_education, cat_idx_marriage, pay_state_ids, pay_severities, num_values, emb_sex, emb_education, emb_marriage, pay_state_table, sev_W, sev_b, num_feat_table, val_W, val_b, pos_table, cls_token, ln_g, ln_b):
    raise NotImplementedError("write your pallas kernel here")


# speedup vs baseline: 1.8935x; 1.8935x over previous
import jax
import jax.numpy as jnp
from jax.experimental import pallas as pl
from jax.experimental.pallas import tpu as pltpu

D = 64
N_CAT = 3
N_PAY = 6
N_NUM = 14
N_TOK = 24
BR = 512


def _ln(v, g, b):
    mu = jnp.mean(v, axis=-1, keepdims=True)
    var = jnp.mean((v - mu) ** 2, axis=-1, keepdims=True)
    return (v - mu) * jax.lax.rsqrt(var + 1e-5) * g + b


def _body(ints_ref, s_ref, emb_sex_ref, emb_edu_ref, emb_mar_ref,
          pay_tab_ref, sev_w_ref, sev_b_ref, num_tab_ref, val_w_ref,
          val_b_ref, pos_ref, cls_ref, g_ref, b_ref, out_ref):
    g = g_ref[:]
    b = b_ref[:]
    pos = pos_ref[:]
    cls_tok = _ln(cls_ref[0, 0, :] + pos[0], g, b)
    out_ref[:, 0, :] = jnp.broadcast_to(cls_tok[None, :], (BR, D))

    tables = (emb_sex_ref[:], emb_edu_ref[:], emb_mar_ref[:])
    for r in range(N_CAT):
        idx = ints_ref[r, 0, 0, :]
        vocab = tables[r].shape[0]
        oh = (idx[:, None] == jax.lax.broadcasted_iota(jnp.int32, (BR, vocab), 1))
        emb = jnp.dot(oh.astype(jnp.float32), tables[r],
                      preferred_element_type=jnp.float32)
        out_ref[:, 1 + r, :] = _ln(emb + pos[1 + r], g, b)

    sev_w = sev_w_ref[:, 0]
    sev_b = sev_b_ref[:]
    pay_tab = pay_tab_ref[:]
    for j in range(N_PAY):
        idx = ints_ref[N_CAT + j, 0, 0, :]
        oh = (idx[:, None] == jax.lax.broadcasted_iota(jnp.int32, (BR, 4), 1))
        st = jnp.dot(oh.astype(jnp.float32), pay_tab,
                     preferred_element_type=jnp.float32)
        sev = s_ref[j, 0, 0, :]
        tok = st + sev[:, None] * sev_w[None, :] + sev_b + pos[4 + j]
        out_ref[:, 1 + N_CAT + j, :] = _ln(tok, g, b)

    val_w = val_w_ref[:, 0]
    val_b = val_b_ref[:]
    num_tab = num_tab_ref[:]
    for j in range(N_NUM):
        x = s_ref[N_PAY + j, 0, 0, :]
        tok = x[:, None] * val_w[None, :] + val_b + num_tab[j] + pos[10 + j]
        out_ref[:, 1 + N_CAT + N_PAY + j, :] = _ln(tok, g, b)


def kernel(cat_idx_sex, cat_idx_education, cat_idx_marriage, pay_state_ids,
           pay_severities, num_values, emb_sex, emb_education, emb_marriage,
           pay_state_table, sev_W, sev_b, num_feat_table, val_W, val_b,
           pos_table, cls_token, ln_g, ln_b):
    B = num_values.shape[0]
    nb = B // BR
    ints = jnp.concatenate([
        cat_idx_sex.astype(jnp.int32)[None, :],
        cat_idx_education.astype(jnp.int32)[None, :],
        cat_idx_marriage.astype(jnp.int32)[None, :],
        pay_state_ids.astype(jnp.int32).T,
    ], axis=0).reshape(N_CAT + N_PAY, nb, 1, BR)
    s_all = jnp.concatenate([pay_severities.T, num_values.T],
                            axis=0).reshape(N_PAY + N_NUM, nb, 1, BR)

    full = lambda shape: pl.BlockSpec(shape, lambda i: (0,) * len(shape))
    grid_spec = pl.GridSpec(
        grid=(nb,),
        in_specs=[
            pl.BlockSpec((N_CAT + N_PAY, 1, 1, BR), lambda i: (0, i, 0, 0)),
            pl.BlockSpec((N_PAY + N_NUM, 1, 1, BR), lambda i: (0, i, 0, 0)),
            full((2, D)), full((7, D)), full((4, D)), full((4, D)),
            full((D, 1)), full((D,)), full((N_NUM, D)), full((D, 1)),
            full((D,)), full((N_TOK, D)), full((1, 1, D)), full((D,)),
            full((D,)),
        ],
        out_specs=pl.BlockSpec((BR, N_TOK, D), lambda i: (i, 0, 0)),
    )
    return pl.pallas_call(
        _body,
        grid_spec=grid_spec,
        out_shape=jax.ShapeDtypeStruct((B, N_TOK, D), jnp.float32),
    )(ints, s_all, emb_sex, emb_education, emb_marriage, pay_state_table,
      sev_W, sev_b, num_feat_table, val_W, val_b, pos_table, cls_token,
      ln_g, ln_b)
